# baseline (speedup 1.0000x reference)
import functools

import jax
import jax.numpy as jnp
from jax import lax
from jax.experimental import pallas as pl
from jax.experimental.pallas import tpu as pltpu
from jax.experimental.pallas import tpu_sc as plsc

N_NODES = 10000
N_EDGES = 320000
IN_DIMS = 128
OUT_DIMS = 64
EPS = 1e-5

D = 80
NTILES = 16
CHUNK = 128
NCHUNK = 160
NBUF = 2
PER_TILE = CHUNK * NCHUNK
E_PAD = PER_TILE * NTILES
ROWS_PT = 632
ROWS = ROWS_PT * NTILES
DUMMY_DST = N_NODES


def _pre_body(x_ref, wl_ref, yp_ref, yn_ref):
    xw = jnp.dot(x_ref[...], wl_ref[...], preferred_element_type=jnp.float32)
    r = x_ref.shape[0]
    col = lax.broadcasted_iota(jnp.int32, (r, D - OUT_DIMS), 1)
    tail = jnp.where(col == 0, 1.0, 0.0).astype(jnp.float32)
    yp_ref[...] = jnp.concatenate([xw[:, :OUT_DIMS], tail], axis=1)
    yn_ref[...] = jnp.concatenate([xw[:, OUT_DIMS:], tail], axis=1)


def _pre(x, wl_cat):
    return pl.pallas_call(
        _pre_body,
        out_shape=[
            jax.ShapeDtypeStruct((N_NODES, D), jnp.float32),
            jax.ShapeDtypeStruct((N_NODES, D), jnp.float32),
        ],
        grid=(5,),
        in_specs=[
            pl.BlockSpec((N_NODES // 5, IN_DIMS), lambda i: (i, 0)),
            pl.BlockSpec((IN_DIMS, 2 * OUT_DIMS), lambda i: (0, 0)),
        ],
        out_specs=[
            pl.BlockSpec((N_NODES // 5, D), lambda i: (i, 0)),
            pl.BlockSpec((N_NODES // 5, D), lambda i: (i, 0)),
        ],
    )(x, wl_cat)


def _sc_body(yp, yn, psrc, pdst, nsrc, ndst, zeros_hbm, outp, outn,
             src_v, dst_v, buf3d, acc, gsem):
    c = lax.axis_index("c")
    s = lax.axis_index("s")
    bufs = [buf3d.at[u] for u in range(NBUF)]

    pltpu.sync_copy(zeros_hbm.at[pl.ds(s * ROWS_PT, ROWS_PT)],
                    acc.at[pl.ds(s * ROWS_PT, ROWS_PT)])
    plsc.subcore_barrier()

    def run(y_h, src_h, dst_h, out_h):
        pltpu.sync_copy(src_h.at[s], src_v)
        pltpu.sync_copy(dst_h.at[s], dst_v)

        pltpu.async_copy(y_h.at[src_v.at[0]], bufs[0], gsem.at[0])

        def body(jo, carry):
            for u in range(NBUF):
                j = jo * NBUF + u
                un = (u + 1) % NBUF

                @pl.when(j + 1 < NCHUNK)
                def _():
                    pltpu.async_copy(
                        y_h.at[src_v.at[j + 1]], bufs[un], gsem.at[un])

                pltpu.make_async_copy(
                    y_h.at[src_v.at[j]], bufs[u], gsem.at[u]).wait()
                pltpu.sync_copy(bufs[u], acc.at[dst_v.at[j]], add=True)
            return carry

        lax.fori_loop(0, NCHUNK // NBUF, body, 0)

        plsc.subcore_barrier()
        pltpu.sync_copy(acc.at[pl.ds(s * ROWS_PT, ROWS_PT)],
                        out_h.at[pl.ds(s * ROWS_PT, ROWS_PT)])

    @pl.when(c == 0)
    def _():
        run(yp, psrc, pdst, outp)

    @pl.when(c == 1)
    def _():
        run(yn, nsrc, ndst, outn)


_sc_agg = functools.partial(
    pl.kernel,
    _sc_body,
    out_type=[
        jax.ShapeDtypeStruct((ROWS, D), jnp.float32),
        jax.ShapeDtypeStruct((ROWS, D), jnp.float32),
    ],
    mesh=plsc.VectorSubcoreMesh(core_axis_name="c", subcore_axis_name="s"),
    compiler_params=pltpu.CompilerParams(use_tc_tiling_on_sc=False),
    scratch_types=[
        pltpu.VMEM((NCHUNK, CHUNK), jnp.int32),
        pltpu.VMEM((NCHUNK, CHUNK), jnp.int32),
        pltpu.VMEM((NBUF, CHUNK, D), jnp.float32),
        pltpu.VMEM_SHARED((ROWS, D), jnp.float32),
        pltpu.SemaphoreType.DMA((NBUF,)),
    ],
)()


def _post_body(x_ref, sp_ref, sn_ref, wr_ref, b_ref, g_ref, be_ref, out_ref):
    xr = jnp.dot(x_ref[...], wr_ref[...], preferred_element_type=jnp.float32)
    sp = sp_ref[...]
    sn = sn_ref[...]
    aggp = sp[:, :OUT_DIMS] / jnp.maximum(sp[:, OUT_DIMS:OUT_DIMS + 1], 1.0)
    aggn = sn[:, :OUT_DIMS] / jnp.maximum(sn[:, OUT_DIMS:OUT_DIMS + 1], 1.0)
    pre = jnp.concatenate([aggp, aggn], axis=1) + xr + b_ref[...]
    mu = jnp.mean(pre, axis=0, keepdims=True)
    var = jnp.mean(jnp.square(pre - mu), axis=0, keepdims=True)
    out = (pre - mu) * lax.rsqrt(var + EPS) * g_ref[...] + be_ref[...]
    out_ref[...] = jnp.maximum(out, 0.0)


def _post(x, sp, sn, wr_cat, b_cat, g_cat, be_cat):
    return pl.pallas_call(
        _post_body,
        out_shape=jax.ShapeDtypeStruct((N_NODES, 2 * OUT_DIMS), jnp.float32),
    )(x, sp, sn, wr_cat, b_cat, g_cat, be_cat)


def _prep_edges(edge_index):
    src = edge_index[0].astype(jnp.int32)
    dst = edge_index[1].astype(jnp.int32)
    pad = E_PAD - N_EDGES
    src = jnp.concatenate([src, jnp.zeros((pad,), jnp.int32)])
    dst = jnp.concatenate([dst, jnp.full((pad,), DUMMY_DST, jnp.int32)])
    return (src.reshape(NTILES, NCHUNK, CHUNK),
            dst.reshape(NTILES, NCHUNK, CHUNK))


def kernel(x, pos_edge_index, neg_edge_index, W_pos_l, W_pos_r, b_pos,
           W_neg_l, W_neg_r, b_neg, gamma, beta):
    psrc, pdst = _prep_edges(pos_edge_index)
    nsrc, ndst = _prep_edges(neg_edge_index)
    wl_cat = jnp.concatenate([W_pos_l, W_neg_l], axis=1)
    wr_cat = jnp.concatenate([W_pos_r, W_neg_r], axis=1)
    b_cat = jnp.concatenate([b_pos, b_neg]).reshape(1, 2 * OUT_DIMS)
    g_cat = gamma.reshape(1, 2 * OUT_DIMS)
    be_cat = beta.reshape(1, 2 * OUT_DIMS)
    zeros_hbm = jnp.zeros((ROWS, D), jnp.float32)

    yp, yn = _pre(x, wl_cat)
    sp_full, sn_full = _sc_agg(yp, yn, psrc, pdst, nsrc, ndst, zeros_hbm)
    sp = sp_full[:N_NODES]
    sn = sn_full[:N_NODES]
    return _post(x, sp, sn, wr_cat, b_cat, g_cat, be_cat)


# speedup vs baseline: 1.6675x; 1.6675x over previous
import functools

import jax
import jax.numpy as jnp
from jax import lax
from jax.experimental import pallas as pl
from jax.experimental.pallas import tpu as pltpu
from jax.experimental.pallas import tpu_sc as plsc

N_NODES = 10000
N_EDGES = 320000
IN_DIMS = 128
OUT_DIMS = 64
EPS = 1e-5

D = 80
NTILES = 16
CHUNK = 128
NROWS_E = N_EDGES // CHUNK
BASE_CH = NROWS_E // NTILES
LAST_EXTRA = NROWS_E - BASE_CH * NTILES
MAX_CH = BASE_CH + LAST_EXTRA
ROWS_PT = 632
ROWS = ROWS_PT * NTILES


def _pre_body(x_ref, wl_ref, yp_ref, yn_ref):
    xw = jnp.dot(x_ref[...], wl_ref[...], preferred_element_type=jnp.float32)
    r = x_ref.shape[0]
    col = lax.broadcasted_iota(jnp.int32, (r, D - OUT_DIMS), 1)
    tail = jnp.where(col == 0, 1.0, 0.0).astype(jnp.float32)
    yp_ref[...] = jnp.concatenate([xw[:, :OUT_DIMS], tail], axis=1)
    yn_ref[...] = jnp.concatenate([xw[:, OUT_DIMS:], tail], axis=1)


def _pre(x, wl_cat):
    return pl.pallas_call(
        _pre_body,
        out_shape=[
            jax.ShapeDtypeStruct((N_NODES, D), jnp.float32),
            jax.ShapeDtypeStruct((N_NODES, D), jnp.float32),
        ],
        grid=(5,),
        in_specs=[
            pl.BlockSpec((N_NODES // 5, IN_DIMS), lambda i: (i, 0)),
            pl.BlockSpec((IN_DIMS, 2 * OUT_DIMS), lambda i: (0, 0)),
        ],
        out_specs=[
            pl.BlockSpec((N_NODES // 5, D), lambda i: (i, 0)),
            pl.BlockSpec((N_NODES // 5, D), lambda i: (i, 0)),
        ],
    )(x, wl_cat)


def _sc_body(yp, yn, pe, ne, zeros_hbm, outp, outn,
             src_v, dst_v, buf, acc, sem):
    c = lax.axis_index("c")
    s = lax.axis_index("s")

    pltpu.sync_copy(zeros_hbm.at[pl.ds(s * ROWS_PT, ROWS_PT)],
                    acc.at[pl.ds(s * ROWS_PT, ROWS_PT)])
    plsc.subcore_barrier()

    nchunk = jnp.where(s == NTILES - 1, MAX_CH, BASE_CH)

    def run(y_h, e_h, out_h):
        pltpu.sync_copy(e_h.at[0, pl.ds(s * BASE_CH, BASE_CH)],
                        src_v.at[pl.ds(0, BASE_CH)])
        pltpu.sync_copy(e_h.at[1, pl.ds(s * BASE_CH, BASE_CH)],
                        dst_v.at[pl.ds(0, BASE_CH)])

        @pl.when(s == NTILES - 1)
        def _():
            pltpu.sync_copy(
                e_h.at[0, pl.ds(NTILES * BASE_CH, LAST_EXTRA)],
                src_v.at[pl.ds(BASE_CH, LAST_EXTRA)])
            pltpu.sync_copy(
                e_h.at[1, pl.ds(NTILES * BASE_CH, LAST_EXTRA)],
                dst_v.at[pl.ds(BASE_CH, LAST_EXTRA)])

        def body(j, carry):
            pltpu.async_copy(y_h.at[src_v.at[j]], buf, sem).wait()
            pltpu.sync_copy(buf, acc.at[dst_v.at[j]], add=True)
            return carry

        lax.fori_loop(0, nchunk, body, 0)
        plsc.subcore_barrier()
        pltpu.sync_copy(acc.at[pl.ds(s * ROWS_PT, ROWS_PT)],
                        out_h.at[pl.ds(s * ROWS_PT, ROWS_PT)])

    @pl.when(c == 0)
    def _():
        run(yp, pe, outp)

    @pl.when(c == 1)
    def _():
        run(yn, ne, outn)


_sc_agg = functools.partial(
    pl.kernel,
    _sc_body,
    out_type=[
        jax.ShapeDtypeStruct((ROWS, D), jnp.float32),
        jax.ShapeDtypeStruct((ROWS, D), jnp.float32),
    ],
    mesh=plsc.VectorSubcoreMesh(core_axis_name="c", subcore_axis_name="s"),
    compiler_params=pltpu.CompilerParams(use_tc_tiling_on_sc=False),
    scratch_types=[
        pltpu.VMEM((MAX_CH, CHUNK), jnp.int32),
        pltpu.VMEM((MAX_CH, CHUNK), jnp.int32),
        pltpu.VMEM((CHUNK, D), jnp.float32),
        pltpu.VMEM_SHARED((ROWS, D), jnp.float32),
        pltpu.SemaphoreType.DMA,
    ],
)()


def _post_body(x_ref, sp_ref, sn_ref, wr_ref, b_ref, g_ref, be_ref, out_ref):
    xr = jnp.dot(x_ref[...], wr_ref[...], preferred_element_type=jnp.float32)
    sp = sp_ref[...]
    sn = sn_ref[...]
    aggp = sp[:, :OUT_DIMS] / jnp.maximum(sp[:, OUT_DIMS:OUT_DIMS + 1], 1.0)
    aggn = sn[:, :OUT_DIMS] / jnp.maximum(sn[:, OUT_DIMS:OUT_DIMS + 1], 1.0)
    pre = jnp.concatenate([aggp, aggn], axis=1) + xr + b_ref[...]
    mu = jnp.mean(pre, axis=0, keepdims=True)
    var = jnp.mean(jnp.square(pre - mu), axis=0, keepdims=True)
    out = (pre - mu) * lax.rsqrt(var + EPS) * g_ref[...] + be_ref[...]
    out_ref[...] = jnp.maximum(out, 0.0)


def _post(x, sp, sn, wr_cat, b_cat, g_cat, be_cat):
    return pl.pallas_call(
        _post_body,
        out_shape=jax.ShapeDtypeStruct((N_NODES, 2 * OUT_DIMS), jnp.float32),
    )(x, sp, sn, wr_cat, b_cat, g_cat, be_cat)


def kernel(x, pos_edge_index, neg_edge_index, W_pos_l, W_pos_r, b_pos,
           W_neg_l, W_neg_r, b_neg, gamma, beta):
    pe = pos_edge_index.astype(jnp.int32).reshape(2, NROWS_E, CHUNK)
    ne = neg_edge_index.astype(jnp.int32).reshape(2, NROWS_E, CHUNK)
    wl_cat = jnp.concatenate([W_pos_l, W_neg_l], axis=1)
    wr_cat = jnp.concatenate([W_pos_r, W_neg_r], axis=1)
    b_cat = jnp.concatenate([b_pos, b_neg]).reshape(1, 2 * OUT_DIMS)
    g_cat = gamma.reshape(1, 2 * OUT_DIMS)
    be_cat = beta.reshape(1, 2 * OUT_DIMS)
    zeros_hbm = jnp.zeros((ROWS, D), jnp.float32)

    yp, yn = _pre(x, wl_cat)
    sp_full, sn_full = _sc_agg(yp, yn, pe, ne, zeros_hbm)
    sp = sp_full[:N_NODES]
    sn = sn_full[:N_NODES]
    return _post(x, sp, sn, wr_cat, b_cat, g_cat, be_cat)


# speedup vs baseline: 1.9740x; 1.1839x over previous
import functools

import jax
import jax.numpy as jnp
from jax import lax
from jax.experimental import pallas as pl
from jax.experimental.pallas import tpu as pltpu
from jax.experimental.pallas import tpu_sc as plsc

N_NODES = 10000
N_EDGES = 320000
IN_DIMS = 128
OUT_DIMS = 64
EPS = 1e-5

D = 80
NTILES = 16
CHUNK = 128
NROWS_E = N_EDGES // CHUNK
BASE_CH = NROWS_E // NTILES
LAST_EXTRA = NROWS_E - BASE_CH * NTILES
MAX_CH = BASE_CH + LAST_EXTRA
GRP = 2
GRP_E = GRP * CHUNK
PT_E = MAX_CH * CHUNK
BASE_E = BASE_CH * CHUNK
ROWS_PT = 632
ROWS = ROWS_PT * NTILES


def _pre_body(x_ref, wl_ref, yp_ref, yn_ref):
    xw = jnp.dot(x_ref[...], wl_ref[...], preferred_element_type=jnp.float32)
    r = x_ref.shape[0]
    col = lax.broadcasted_iota(jnp.int32, (r, D - OUT_DIMS), 1)
    tail = jnp.where(col == 0, 1.0, 0.0).astype(jnp.float32)
    yp_ref[...] = jnp.concatenate([xw[:, :OUT_DIMS], tail], axis=1)
    yn_ref[...] = jnp.concatenate([xw[:, OUT_DIMS:], tail], axis=1)


def _pre(x, wl_cat):
    return pl.pallas_call(
        _pre_body,
        out_shape=[
            jax.ShapeDtypeStruct((N_NODES, D), jnp.float32),
            jax.ShapeDtypeStruct((N_NODES, D), jnp.float32),
        ],
        grid=(5,),
        in_specs=[
            pl.BlockSpec((N_NODES // 5, IN_DIMS), lambda i: (i, 0)),
            pl.BlockSpec((IN_DIMS, 2 * OUT_DIMS), lambda i: (0, 0)),
        ],
        out_specs=[
            pl.BlockSpec((N_NODES // 5, D), lambda i: (i, 0)),
            pl.BlockSpec((N_NODES // 5, D), lambda i: (i, 0)),
        ],
    )(x, wl_cat)


def _sc_body(yp, yn, pe, ne, zeros_hbm, outp, outn,
             src_v, dst_v, buf, acc, sem):
    c = lax.axis_index("c")
    s = lax.axis_index("s")

    pltpu.sync_copy(zeros_hbm.at[pl.ds(s * ROWS_PT, ROWS_PT)],
                    acc.at[pl.ds(s * ROWS_PT, ROWS_PT)])
    plsc.subcore_barrier()

    nchunk = jnp.where(s == NTILES - 1, MAX_CH, BASE_CH)

    def run(y_h, e_h, out_h):
        pltpu.sync_copy(e_h.at[0, pl.ds(s * BASE_E, BASE_E)],
                        src_v.at[pl.ds(0, BASE_E)])
        pltpu.sync_copy(e_h.at[1, pl.ds(s * BASE_E, BASE_E)],
                        dst_v.at[pl.ds(0, BASE_E)])

        @pl.when(s == NTILES - 1)
        def _():
            pltpu.sync_copy(
                e_h.at[0, pl.ds(NTILES * BASE_E, PT_E - BASE_E)],
                src_v.at[pl.ds(BASE_E, PT_E - BASE_E)])
            pltpu.sync_copy(
                e_h.at[1, pl.ds(NTILES * BASE_E, PT_E - BASE_E)],
                dst_v.at[pl.ds(BASE_E, PT_E - BASE_E)])

        def body(g, carry):
            pltpu.async_copy(
                y_h.at[src_v.at[pl.ds(g * GRP_E, GRP_E)]], buf, sem).wait()
            pltpu.sync_copy(
                buf, acc.at[dst_v.at[pl.ds(g * GRP_E, GRP_E)]], add=True)
            return carry

        lax.fori_loop(0, nchunk // GRP, body, 0)
        plsc.subcore_barrier()
        pltpu.sync_copy(acc.at[pl.ds(s * ROWS_PT, ROWS_PT)],
                        out_h.at[pl.ds(s * ROWS_PT, ROWS_PT)])

    @pl.when(c == 0)
    def _():
        run(yp, pe, outp)

    @pl.when(c == 1)
    def _():
        run(yn, ne, outn)


_sc_agg = functools.partial(
    pl.kernel,
    _sc_body,
    out_type=[
        jax.ShapeDtypeStruct((ROWS, D), jnp.float32),
        jax.ShapeDtypeStruct((ROWS, D), jnp.float32),
    ],
    mesh=plsc.VectorSubcoreMesh(core_axis_name="c", subcore_axis_name="s"),
    compiler_params=pltpu.CompilerParams(use_tc_tiling_on_sc=False),
    scratch_types=[
        pltpu.VMEM((PT_E,), jnp.int32),
        pltpu.VMEM((PT_E,), jnp.int32),
        pltpu.VMEM((GRP_E, D), jnp.float32),
        pltpu.VMEM_SHARED((ROWS, D), jnp.float32),
        pltpu.SemaphoreType.DMA,
    ],
)()


def _post_body(x_ref, sp_ref, sn_ref, wr_ref, b_ref, g_ref, be_ref, out_ref):
    xr = jnp.dot(x_ref[...], wr_ref[...], preferred_element_type=jnp.float32)
    sp = sp_ref[...]
    sn = sn_ref[...]
    aggp = sp[:, :OUT_DIMS] / jnp.maximum(sp[:, OUT_DIMS:OUT_DIMS + 1], 1.0)
    aggn = sn[:, :OUT_DIMS] / jnp.maximum(sn[:, OUT_DIMS:OUT_DIMS + 1], 1.0)
    pre = jnp.concatenate([aggp, aggn], axis=1) + xr + b_ref[...]
    mu = jnp.mean(pre, axis=0, keepdims=True)
    var = jnp.mean(jnp.square(pre - mu), axis=0, keepdims=True)
    out = (pre - mu) * lax.rsqrt(var + EPS) * g_ref[...] + be_ref[...]
    out_ref[...] = jnp.maximum(out, 0.0)


def _post(x, sp, sn, wr_cat, b_cat, g_cat, be_cat):
    return pl.pallas_call(
        _post_body,
        out_shape=jax.ShapeDtypeStruct((N_NODES, 2 * OUT_DIMS), jnp.float32),
    )(x, sp, sn, wr_cat, b_cat, g_cat, be_cat)


def kernel(x, pos_edge_index, neg_edge_index, W_pos_l, W_pos_r, b_pos,
           W_neg_l, W_neg_r, b_neg, gamma, beta):
    pe = pos_edge_index.astype(jnp.int32)
    ne = neg_edge_index.astype(jnp.int32)
    wl_cat = jnp.concatenate([W_pos_l, W_neg_l], axis=1)
    wr_cat = jnp.concatenate([W_pos_r, W_neg_r], axis=1)
    b_cat = jnp.concatenate([b_pos, b_neg]).reshape(1, 2 * OUT_DIMS)
    g_cat = gamma.reshape(1, 2 * OUT_DIMS)
    be_cat = beta.reshape(1, 2 * OUT_DIMS)
    zeros_hbm = jnp.zeros((ROWS, D), jnp.float32)

    yp, yn = _pre(x, wl_cat)
    sp_full, sn_full = _sc_agg(yp, yn, pe, ne, zeros_hbm)
    sp = sp_full[:N_NODES]
    sn = sn_full[:N_NODES]
    return _post(x, sp, sn, wr_cat, b_cat, g_cat, be_cat)


# speedup vs baseline: 2.1836x; 1.1062x over previous
import functools

import jax
import jax.numpy as jnp
from jax import lax
from jax.experimental import pallas as pl
from jax.experimental.pallas import tpu as pltpu
from jax.experimental.pallas import tpu_sc as plsc

N_NODES = 10000
N_EDGES = 320000
IN_DIMS = 128
OUT_DIMS = 64
EPS = 1e-5

D = 80
NTILES = 16
CHUNK = 128
EPT = N_EDGES // NTILES
NSEG = 5
SEG_E = EPT // NSEG
GRP_E = 800
GPS = SEG_E // GRP_E
ROWS_PT = 632
ROWS = ROWS_PT * NTILES


def _pre_body(x_ref, wl_ref, yp_ref, yn_ref):
    xw = jnp.dot(x_ref[...], wl_ref[...], preferred_element_type=jnp.float32)
    r = x_ref.shape[0]
    col = lax.broadcasted_iota(jnp.int32, (r, D - OUT_DIMS), 1)
    tail = jnp.where(col == 0, 1.0, 0.0).astype(jnp.float32)
    yp_ref[...] = jnp.concatenate([xw[:, :OUT_DIMS], tail], axis=1)
    yn_ref[...] = jnp.concatenate([xw[:, OUT_DIMS:], tail], axis=1)


def _pre(x, wl_cat):
    return pl.pallas_call(
        _pre_body,
        out_shape=[
            jax.ShapeDtypeStruct((N_NODES, D), jnp.float32),
            jax.ShapeDtypeStruct((N_NODES, D), jnp.float32),
        ],
        grid=(5,),
        in_specs=[
            pl.BlockSpec((N_NODES // 5, IN_DIMS), lambda i: (i, 0)),
            pl.BlockSpec((IN_DIMS, 2 * OUT_DIMS), lambda i: (0, 0)),
        ],
        out_specs=[
            pl.BlockSpec((N_NODES // 5, D), lambda i: (i, 0)),
            pl.BlockSpec((N_NODES // 5, D), lambda i: (i, 0)),
        ],
    )(x, wl_cat)


def _sc_body(yp, yn, pe, ne, zeros_hbm, outp, outn,
             src_v, dst_v, buf, acc, sem):
    c = lax.axis_index("c")
    s = lax.axis_index("s")

    pltpu.sync_copy(zeros_hbm.at[pl.ds(s * ROWS_PT, ROWS_PT)],
                    acc.at[pl.ds(s * ROWS_PT, ROWS_PT)])
    plsc.subcore_barrier()

    def run(y_h, e_h, out_h):
        def seg_body(k, carry):
            off = s * EPT + k * SEG_E
            pltpu.sync_copy(e_h.at[0, pl.ds(off, SEG_E)], src_v)
            pltpu.sync_copy(e_h.at[1, pl.ds(off, SEG_E)], dst_v)

            def body(g, c2):
                pltpu.async_copy(
                    y_h.at[src_v.at[pl.ds(g * GRP_E, GRP_E)]], buf, sem
                ).wait()
                pltpu.sync_copy(
                    buf, acc.at[dst_v.at[pl.ds(g * GRP_E, GRP_E)]], add=True)
                return c2

            lax.fori_loop(0, GPS, body, 0)
            return carry

        lax.fori_loop(0, NSEG, seg_body, 0)
        plsc.subcore_barrier()
        pltpu.sync_copy(acc.at[pl.ds(s * ROWS_PT, ROWS_PT)],
                        out_h.at[pl.ds(s * ROWS_PT, ROWS_PT)])

    @pl.when(c == 0)
    def _():
        run(yp, pe, outp)

    @pl.when(c == 1)
    def _():
        run(yn, ne, outn)


_sc_agg = functools.partial(
    pl.kernel,
    _sc_body,
    out_type=[
        jax.ShapeDtypeStruct((ROWS, D), jnp.float32),
        jax.ShapeDtypeStruct((ROWS, D), jnp.float32),
    ],
    mesh=plsc.VectorSubcoreMesh(core_axis_name="c", subcore_axis_name="s"),
    compiler_params=pltpu.CompilerParams(use_tc_tiling_on_sc=False),
    scratch_types=[
        pltpu.VMEM((SEG_E,), jnp.int32),
        pltpu.VMEM((SEG_E,), jnp.int32),
        pltpu.VMEM((GRP_E, D), jnp.float32),
        pltpu.VMEM_SHARED((ROWS, D), jnp.float32),
        pltpu.SemaphoreType.DMA,
    ],
)()


def _post_body(x_ref, sp_ref, sn_ref, wr_ref, b_ref, g_ref, be_ref, out_ref):
    xr = jnp.dot(x_ref[...], wr_ref[...], preferred_element_type=jnp.float32)
    sp = sp_ref[...]
    sn = sn_ref[...]
    aggp = sp[:, :OUT_DIMS] / jnp.maximum(sp[:, OUT_DIMS:OUT_DIMS + 1], 1.0)
    aggn = sn[:, :OUT_DIMS] / jnp.maximum(sn[:, OUT_DIMS:OUT_DIMS + 1], 1.0)
    pre = jnp.concatenate([aggp, aggn], axis=1) + xr + b_ref[...]
    mu = jnp.mean(pre, axis=0, keepdims=True)
    var = jnp.mean(jnp.square(pre - mu), axis=0, keepdims=True)
    out = (pre - mu) * lax.rsqrt(var + EPS) * g_ref[...] + be_ref[...]
    out_ref[...] = jnp.maximum(out, 0.0)


def _post(x, sp, sn, wr_cat, b_cat, g_cat, be_cat):
    return pl.pallas_call(
        _post_body,
        out_shape=jax.ShapeDtypeStruct((N_NODES, 2 * OUT_DIMS), jnp.float32),
    )(x, sp, sn, wr_cat, b_cat, g_cat, be_cat)


def kernel(x, pos_edge_index, neg_edge_index, W_pos_l, W_pos_r, b_pos,
           W_neg_l, W_neg_r, b_neg, gamma, beta):
    pe = pos_edge_index.astype(jnp.int32)
    ne = neg_edge_index.astype(jnp.int32)
    wl_cat = jnp.concatenate([W_pos_l, W_neg_l], axis=1)
    wr_cat = jnp.concatenate([W_pos_r, W_neg_r], axis=1)
    b_cat = jnp.concatenate([b_pos, b_neg]).reshape(1, 2 * OUT_DIMS)
    g_cat = gamma.reshape(1, 2 * OUT_DIMS)
    be_cat = beta.reshape(1, 2 * OUT_DIMS)
    zeros_hbm = jnp.zeros((ROWS, D), jnp.float32)

    yp, yn = _pre(x, wl_cat)
    sp_full, sn_full = _sc_agg(yp, yn, pe, ne, zeros_hbm)
    sp = sp_full[:N_NODES]
    sn = sn_full[:N_NODES]
    return _post(x, sp, sn, wr_cat, b_cat, g_cat, be_cat)


# speedup vs baseline: 2.2041x; 1.0094x over previous
import functools

import jax
import jax.numpy as jnp
from jax import lax
from jax.experimental import pallas as pl
from jax.experimental.pallas import tpu as pltpu
from jax.experimental.pallas import tpu_sc as plsc

N_NODES = 10000
N_EDGES = 320000
IN_DIMS = 128
OUT_DIMS = 64
EPS = 1e-5

D = 80
NTILES = 16
CHUNK = 128
EPT = N_EDGES // NTILES
NSEG = 5
SEG_E = EPT // NSEG
GRP_E = 400
GPS = SEG_E // GRP_E
ROWS_PT = 632
ROWS = ROWS_PT * NTILES


def _pre_body(x_ref, wl_ref, yp_ref, yn_ref):
    xw = jnp.dot(x_ref[...], wl_ref[...], preferred_element_type=jnp.float32)
    r = x_ref.shape[0]
    col = lax.broadcasted_iota(jnp.int32, (r, D - OUT_DIMS), 1)
    tail = jnp.where(col == 0, 1.0, 0.0).astype(jnp.float32)
    yp_ref[...] = jnp.concatenate([xw[:, :OUT_DIMS], tail], axis=1)
    yn_ref[...] = jnp.concatenate([xw[:, OUT_DIMS:], tail], axis=1)


def _pre(x, wl_cat):
    return pl.pallas_call(
        _pre_body,
        out_shape=[
            jax.ShapeDtypeStruct((N_NODES, D), jnp.float32),
            jax.ShapeDtypeStruct((N_NODES, D), jnp.float32),
        ],
        grid=(5,),
        in_specs=[
            pl.BlockSpec((N_NODES // 5, IN_DIMS), lambda i: (i, 0)),
            pl.BlockSpec((IN_DIMS, 2 * OUT_DIMS), lambda i: (0, 0)),
        ],
        out_specs=[
            pl.BlockSpec((N_NODES // 5, D), lambda i: (i, 0)),
            pl.BlockSpec((N_NODES // 5, D), lambda i: (i, 0)),
        ],
    )(x, wl_cat)


def _sc_body(yp, yn, pe, ne, zeros_hbm, outp, outn,
             src_v, dst_v, buf_a, buf_b, acc, sem_a, sem_b, sem_sa, sem_sb):
    c = lax.axis_index("c")
    s = lax.axis_index("s")

    pltpu.sync_copy(zeros_hbm.at[pl.ds(s * ROWS_PT, ROWS_PT)],
                    acc.at[pl.ds(s * ROWS_PT, ROWS_PT)])
    plsc.subcore_barrier()

    def run(y_h, e_h, out_h):
        def seg_body(k, carry):
            off = s * EPT + k * SEG_E
            pltpu.sync_copy(e_h.at[0, pl.ds(off, SEG_E)], src_v)
            pltpu.sync_copy(e_h.at[1, pl.ds(off, SEG_E)], dst_v)

            def body(p, c2):
                i0 = pl.ds((2 * p) * GRP_E, GRP_E)
                i1 = pl.ds((2 * p + 1) * GRP_E, GRP_E)
                g0 = pltpu.async_copy(y_h.at[src_v.at[i0]], buf_a, sem_a)
                g1 = pltpu.async_copy(y_h.at[src_v.at[i1]], buf_b, sem_b)
                g0.wait()
                s0 = pltpu.async_copy(
                    buf_a, acc.at[dst_v.at[i0]], sem_sa, add=True)
                g1.wait()
                s1 = pltpu.async_copy(
                    buf_b, acc.at[dst_v.at[i1]], sem_sb, add=True)
                s0.wait()
                s1.wait()
                return c2

            lax.fori_loop(0, GPS // 2, body, 0)
            return carry

        lax.fori_loop(0, NSEG, seg_body, 0)
        plsc.subcore_barrier()
        pltpu.sync_copy(acc.at[pl.ds(s * ROWS_PT, ROWS_PT)],
                        out_h.at[pl.ds(s * ROWS_PT, ROWS_PT)])

    @pl.when(c == 0)
    def _():
        run(yp, pe, outp)

    @pl.when(c == 1)
    def _():
        run(yn, ne, outn)


_sc_agg = functools.partial(
    pl.kernel,
    _sc_body,
    out_type=[
        jax.ShapeDtypeStruct((ROWS, D), jnp.float32),
        jax.ShapeDtypeStruct((ROWS, D), jnp.float32),
    ],
    mesh=plsc.VectorSubcoreMesh(core_axis_name="c", subcore_axis_name="s"),
    compiler_params=pltpu.CompilerParams(use_tc_tiling_on_sc=False),
    scratch_types=[
        pltpu.VMEM((SEG_E,), jnp.int32),
        pltpu.VMEM((SEG_E,), jnp.int32),
        pltpu.VMEM((GRP_E, D), jnp.float32),
        pltpu.VMEM((GRP_E, D), jnp.float32),
        pltpu.VMEM_SHARED((ROWS, D), jnp.float32),
        pltpu.SemaphoreType.DMA,
        pltpu.SemaphoreType.DMA,
        pltpu.SemaphoreType.DMA,
        pltpu.SemaphoreType.DMA,
    ],
)()


def _post_body(x_ref, sp_ref, sn_ref, wr_ref, b_ref, g_ref, be_ref, out_ref):
    xr = jnp.dot(x_ref[...], wr_ref[...], preferred_element_type=jnp.float32)
    sp = sp_ref[...]
    sn = sn_ref[...]
    aggp = sp[:, :OUT_DIMS] / jnp.maximum(sp[:, OUT_DIMS:OUT_DIMS + 1], 1.0)
    aggn = sn[:, :OUT_DIMS] / jnp.maximum(sn[:, OUT_DIMS:OUT_DIMS + 1], 1.0)
    pre = jnp.concatenate([aggp, aggn], axis=1) + xr + b_ref[...]
    mu = jnp.mean(pre, axis=0, keepdims=True)
    var = jnp.mean(jnp.square(pre - mu), axis=0, keepdims=True)
    out = (pre - mu) * lax.rsqrt(var + EPS) * g_ref[...] + be_ref[...]
    out_ref[...] = jnp.maximum(out, 0.0)


def _post(x, sp, sn, wr_cat, b_cat, g_cat, be_cat):
    return pl.pallas_call(
        _post_body,
        out_shape=jax.ShapeDtypeStruct((N_NODES, 2 * OUT_DIMS), jnp.float32),
    )(x, sp, sn, wr_cat, b_cat, g_cat, be_cat)


def kernel(x, pos_edge_index, neg_edge_index, W_pos_l, W_pos_r, b_pos,
           W_neg_l, W_neg_r, b_neg, gamma, beta):
    pe = pos_edge_index.astype(jnp.int32)
    ne = neg_edge_index.astype(jnp.int32)
    wl_cat = jnp.concatenate([W_pos_l, W_neg_l], axis=1)
    wr_cat = jnp.concatenate([W_pos_r, W_neg_r], axis=1)
    b_cat = jnp.concatenate([b_pos, b_neg]).reshape(1, 2 * OUT_DIMS)
    g_cat = gamma.reshape(1, 2 * OUT_DIMS)
    be_cat = beta.reshape(1, 2 * OUT_DIMS)
    zeros_hbm = jnp.zeros((ROWS, D), jnp.float32)

    yp, yn = _pre(x, wl_cat)
    sp_full, sn_full = _sc_agg(yp, yn, pe, ne, zeros_hbm)
    sp = sp_full[:N_NODES]
    sn = sn_full[:N_NODES]
    return _post(x, sp, sn, wr_cat, b_cat, g_cat, be_cat)


# speedup vs baseline: 2.4742x; 1.1226x over previous
import functools

import jax
import jax.numpy as jnp
from jax import lax
from jax.experimental import pallas as pl
from jax.experimental.pallas import tpu as pltpu
from jax.experimental.pallas import tpu_sc as plsc

N_NODES = 10000
N_EDGES = 320000
IN_DIMS = 128
OUT_DIMS = 64
EPS = 1e-5

D = OUT_DIMS
NTILES = 16
EPT = N_EDGES // NTILES
NSEG = 5
SEG_E = EPT // NSEG
GRP_E = 800
GPS = SEG_E // GRP_E
VPG = GRP_E // 16
ROWS_PT = 632
ROWS = ROWS_PT * NTILES


def _pre_body(x_ref, wl_ref, yp_ref, yn_ref):
    xw = jnp.dot(x_ref[...], wl_ref[...], preferred_element_type=jnp.float32)
    yp_ref[...] = xw[:, :OUT_DIMS]
    yn_ref[...] = xw[:, OUT_DIMS:]


def _pre(x, wl_cat):
    return pl.pallas_call(
        _pre_body,
        out_shape=[
            jax.ShapeDtypeStruct((N_NODES, D), jnp.float32),
            jax.ShapeDtypeStruct((N_NODES, D), jnp.float32),
        ],
        grid=(5,),
        in_specs=[
            pl.BlockSpec((N_NODES // 5, IN_DIMS), lambda i: (i, 0)),
            pl.BlockSpec((IN_DIMS, 2 * OUT_DIMS), lambda i: (0, 0)),
        ],
        out_specs=[
            pl.BlockSpec((N_NODES // 5, D), lambda i: (i, 0)),
            pl.BlockSpec((N_NODES // 5, D), lambda i: (i, 0)),
        ],
    )(x, wl_cat)


def _sc_body(yp, yn, pe, ne, zeros_hbm, outp, outn, histp, histn,
             src_v, dst_v, buf, hist, acc, sem):
    c = lax.axis_index("c")
    s = lax.axis_index("s")

    pltpu.sync_copy(zeros_hbm.at[pl.ds(s * ROWS_PT, ROWS_PT)],
                    acc.at[pl.ds(s * ROWS_PT, ROWS_PT)])

    zero16 = jnp.zeros((16,), jnp.float32)

    def zbody(i, carry):
        hist[pl.ds(i * 16, 16)] = zero16
        return carry

    lax.fori_loop(0, ROWS // 16, zbody, 0)
    ones16 = jnp.ones((16,), jnp.float32)
    plsc.subcore_barrier()

    def run(y_h, e_h, out_h, hist_h):
        def seg_body(k, carry):
            off = s * EPT + k * SEG_E
            pltpu.sync_copy(e_h.at[0, pl.ds(off, SEG_E)], src_v)
            pltpu.sync_copy(e_h.at[1, pl.ds(off, SEG_E)], dst_v)

            def body(g, c2):
                gat = pltpu.async_copy(
                    y_h.at[src_v.at[pl.ds(g * GRP_E, GRP_E)]], buf, sem)

                def hbody(v, c3):
                    idx16 = dst_v[pl.ds(g * GRP_E + v * 16, 16)]
                    plsc.addupdate_scatter(hist, [idx16], ones16)
                    return c3

                lax.fori_loop(0, VPG, hbody, 0)

                gat.wait()
                pltpu.sync_copy(
                    buf, acc.at[dst_v.at[pl.ds(g * GRP_E, GRP_E)]], add=True)
                return c2

            lax.fori_loop(0, GPS, body, 0)
            return carry

        lax.fori_loop(0, NSEG, seg_body, 0)
        pltpu.sync_copy(hist, hist_h.at[s])
        plsc.subcore_barrier()
        pltpu.sync_copy(acc.at[pl.ds(s * ROWS_PT, ROWS_PT)],
                        out_h.at[pl.ds(s * ROWS_PT, ROWS_PT)])

    @pl.when(c == 0)
    def _():
        run(yp, pe, outp, histp)

    @pl.when(c == 1)
    def _():
        run(yn, ne, outn, histn)


_sc_agg = functools.partial(
    pl.kernel,
    _sc_body,
    out_type=[
        jax.ShapeDtypeStruct((ROWS, D), jnp.float32),
        jax.ShapeDtypeStruct((ROWS, D), jnp.float32),
        jax.ShapeDtypeStruct((NTILES, ROWS), jnp.float32),
        jax.ShapeDtypeStruct((NTILES, ROWS), jnp.float32),
    ],
    mesh=plsc.VectorSubcoreMesh(core_axis_name="c", subcore_axis_name="s"),
    compiler_params=pltpu.CompilerParams(use_tc_tiling_on_sc=False,
                                         needs_layout_passes=False),
    scratch_types=[
        pltpu.VMEM((SEG_E,), jnp.int32),
        pltpu.VMEM((SEG_E,), jnp.int32),
        pltpu.VMEM((GRP_E, D), jnp.float32),
        pltpu.VMEM((ROWS,), jnp.float32),
        pltpu.VMEM_SHARED((ROWS, D), jnp.float32),
        pltpu.SemaphoreType.DMA,
    ],
)()


def _post_body(x_ref, sp_ref, sn_ref, hp_ref, hn_ref, wr_ref, b_ref, g_ref,
               be_ref, out_ref):
    xr = jnp.dot(x_ref[...], wr_ref[...], preferred_element_type=jnp.float32)
    ones_c = jnp.ones((NTILES, 1), jnp.float32)
    dn = (((0,), (0,)), ((), ()))
    cntp = lax.dot_general(hp_ref[...], ones_c, dn,
                           preferred_element_type=jnp.float32)
    cntn = lax.dot_general(hn_ref[...], ones_c, dn,
                           preferred_element_type=jnp.float32)
    aggp = sp_ref[...] / jnp.maximum(cntp[:N_NODES], 1.0)
    aggn = sn_ref[...] / jnp.maximum(cntn[:N_NODES], 1.0)
    pre = jnp.concatenate([aggp, aggn], axis=1) + xr + b_ref[...]
    mu = jnp.mean(pre, axis=0, keepdims=True)
    var = jnp.mean(jnp.square(pre - mu), axis=0, keepdims=True)
    out = (pre - mu) * lax.rsqrt(var + EPS) * g_ref[...] + be_ref[...]
    out_ref[...] = jnp.maximum(out, 0.0)


def _post(x, sp, sn, hp, hn, wr_cat, b_cat, g_cat, be_cat):
    return pl.pallas_call(
        _post_body,
        out_shape=jax.ShapeDtypeStruct((N_NODES, 2 * OUT_DIMS), jnp.float32),
    )(x, sp, sn, hp, hn, wr_cat, b_cat, g_cat, be_cat)


def kernel(x, pos_edge_index, neg_edge_index, W_pos_l, W_pos_r, b_pos,
           W_neg_l, W_neg_r, b_neg, gamma, beta):
    pe = pos_edge_index.astype(jnp.int32)
    ne = neg_edge_index.astype(jnp.int32)
    wl_cat = jnp.concatenate([W_pos_l, W_neg_l], axis=1)
    wr_cat = jnp.concatenate([W_pos_r, W_neg_r], axis=1)
    b_cat = jnp.concatenate([b_pos, b_neg]).reshape(1, 2 * OUT_DIMS)
    g_cat = gamma.reshape(1, 2 * OUT_DIMS)
    be_cat = beta.reshape(1, 2 * OUT_DIMS)
    zeros_hbm = jnp.zeros((ROWS, D), jnp.float32)

    yp, yn = _pre(x, wl_cat)
    sp_full, sn_full, hp, hn = _sc_agg(yp, yn, pe, ne, zeros_hbm)
    sp = sp_full[:N_NODES]
    sn = sn_full[:N_NODES]
    return _post(x, sp, sn, hp, hn, wr_cat, b_cat, g_cat, be_cat)
